# baseline (speedup 1.0000x reference)
import jax
import jax.numpy as jnp
from jax.experimental import pallas as pl
from jax.experimental.pallas import tpu as pltpu

ALPHA = 0.0257
A = 1.0
NORM = 27353.34765625
SRC_INTENSITY = 100000.0 / NORM
FIRE_THRESHOLD = (1000.0 - 20.0) / NORM


def _loss_block_kernel(dt_ref, x_ref, halo_ref, o_ref, op_ref, tg_ref,
                       acc_ref):
    j = pl.program_id(0)
    h = jax.lax.rem(j, 2)
    x = x_ref[0]
    o = o_ref[0]
    op = op_ref[0]
    tg = tg_ref[0]
    halo = halo_ref[0, 0]
    inv_dt = 1.0 / dt_ref[0, 0, 0]

    Dc, H, W = x.shape
    zplane = jnp.zeros((H, W), x.dtype)
    zH = jnp.zeros((Dc, 1, W), x.dtype)
    zW = jnp.zeros((Dc, H, 1), x.dtype)

    below = jnp.where(h == 0, halo, zplane)
    above = jnp.where(h == 1, halo, zplane)
    nbr = jnp.concatenate([x[1:], below[None]], axis=0)
    nbr = nbr + jnp.concatenate([above[None], x[:-1]], axis=0)
    nbr = nbr + jnp.concatenate([x[:, 1:], zH], axis=1)
    nbr = nbr + jnp.concatenate([zH, x[:, :-1]], axis=1)
    nbr = nbr + jnp.concatenate([x[:, :, 1:], zW], axis=2)
    nbr = nbr + jnp.concatenate([zW, x[:, :, :-1]], axis=2)
    lap = nbr - 6.0 * x

    src = jnp.where(x > FIRE_THRESHOLD,
                    jnp.float32(SRC_INTENSITY), jnp.float32(0.0))
    res = (o - op) * inv_dt - ALPHA * lap - src
    diff = o - tg
    tot = res * res + diff * diff

    s = jnp.sum(tot)
    acc_ref[0] = jnp.full((8, 128), s, jnp.float32)


def kernel(input, output, output_past, t, t_past, target):
    B, C, D, H, W = input.shape
    x = input.reshape(B, D, H, W)
    o = output.reshape(B, D, H, W)
    op = output_past.reshape(B, D, H, W)
    tg = target.reshape(B, D, H, W)
    dt = jnp.broadcast_to((t - t_past)[:, :, None], (B, 8, 128))

    half = D // 2
    n_steps = 2 * B

    vol_spec = pl.BlockSpec((1, half, H, W),
                            lambda j: (j // 2, j % 2, 0, 0))
    halo_spec = pl.BlockSpec((1, 1, H, W),
                             lambda j: (j // 2, half - (j % 2), 0, 0))
    dt_spec = pl.BlockSpec((1, 8, 128), lambda j: (j // 2, 0, 0))
    out_spec = pl.BlockSpec((1, 8, 128), lambda j: (j, 0, 0))

    partials = pl.pallas_call(
        _loss_block_kernel,
        grid=(n_steps,),
        in_specs=[dt_spec, vol_spec, halo_spec, vol_spec, vol_spec, vol_spec],
        out_specs=out_spec,
        out_shape=jax.ShapeDtypeStruct((n_steps, 8, 128), jnp.float32),
        compiler_params=pltpu.CompilerParams(
            dimension_semantics=("parallel",),
            vmem_limit_bytes=64 * 1024 * 1024,
        ),
        name="combined_loss_fused",
    )(dt, x, x, o, op, tg)

    n = jnp.float32(B * C * D * H * W)
    return jnp.sum(partials[:, 0, 0]) / n


# speedup vs baseline: 1.0760x; 1.0760x over previous
import jax
import jax.numpy as jnp
from jax.experimental import pallas as pl
from jax.experimental.pallas import tpu as pltpu

ALPHA = 0.0257
A = 1.0
NORM = 27353.34765625
SRC_INTENSITY = 100000.0 / NORM
FIRE_THRESHOLD = (1000.0 - 20.0) / NORM


def _loss_block_kernel(dt_ref, x_ref, o_ref, op_ref, tg_ref, acc_ref):
    x = x_ref[0]
    o = o_ref[0]
    op = op_ref[0]
    tg = tg_ref[0]
    inv_dt = 1.0 / dt_ref[0, 0, 0]

    D, H, W = x.shape
    zD = jnp.zeros((1, H, W), x.dtype)
    zH = jnp.zeros((D, 1, W), x.dtype)
    zW = jnp.zeros((D, H, 1), x.dtype)

    nbr = jnp.concatenate([x[1:], zD], axis=0)
    nbr = nbr + jnp.concatenate([zD, x[:-1]], axis=0)
    nbr = nbr + jnp.concatenate([x[:, 1:], zH], axis=1)
    nbr = nbr + jnp.concatenate([zH, x[:, :-1]], axis=1)
    nbr = nbr + jnp.concatenate([x[:, :, 1:], zW], axis=2)
    nbr = nbr + jnp.concatenate([zW, x[:, :, :-1]], axis=2)
    lap = nbr - 6.0 * x

    src = jnp.where(x > FIRE_THRESHOLD,
                    jnp.float32(SRC_INTENSITY), jnp.float32(0.0))
    res = (o - op) * inv_dt - ALPHA * lap - src
    diff = o - tg
    tot = res * res + diff * diff

    s = jnp.full((8, 128), jnp.sum(tot), jnp.float32)

    @pl.when(pl.program_id(0) == 0)
    def _init():
        acc_ref[...] = s

    @pl.when(pl.program_id(0) != 0)
    def _accum():
        acc_ref[...] = acc_ref[...] + s


def kernel(input, output, output_past, t, t_past, target):
    B, C, D, H, W = input.shape
    x = input.reshape(B, D, H, W)
    o = output.reshape(B, D, H, W)
    op = output_past.reshape(B, D, H, W)
    tg = target.reshape(B, D, H, W)
    dt = jnp.broadcast_to((t - t_past)[:, :, None], (B, 8, 128))

    vol_spec = pl.BlockSpec((1, D, H, W), lambda i: (i, 0, 0, 0))
    dt_spec = pl.BlockSpec((1, 8, 128), lambda i: (i, 0, 0))
    out_spec = pl.BlockSpec((8, 128), lambda i: (0, 0))

    total = pl.pallas_call(
        _loss_block_kernel,
        grid=(B,),
        in_specs=[dt_spec, vol_spec, vol_spec, vol_spec, vol_spec],
        out_specs=out_spec,
        out_shape=jax.ShapeDtypeStruct((8, 128), jnp.float32),
        compiler_params=pltpu.CompilerParams(
            dimension_semantics=("arbitrary",),
            vmem_limit_bytes=64 * 1024 * 1024,
        ),
        name="combined_loss_fused",
    )(dt, x, o, op, tg)

    n = jnp.float32(B * C * D * H * W)
    return total[0, 0] / n
